# baseline (speedup 1.0000x reference)
import functools

import jax
import jax.numpy as jnp
from jax import lax
from jax.experimental import pallas as pl
from jax.experimental.pallas import tpu as pltpu
from jax.experimental.pallas import tpu_sc as plsc

K = 2
BM = 128
SB = 256
NC = 2
NS = 16
NW = NC * NS
CH = 32
CH2 = 16

_INV_SQRT2 = 0.7071067811865476


def _make_gate_body(n, e, nb):
    def body(x_ref, gw_ref, row0_ref, row1_ref, w0_ref, w1_ref,
             be_ref, nv_ref):
        x = x_ref[...]
        logits = jnp.dot(x, gw_ref[...], preferred_element_type=jnp.float32)
        eidx = lax.broadcasted_iota(jnp.int32, (n, e), 1)
        m1 = jnp.max(logits, axis=1, keepdims=True)
        i1 = jnp.min(jnp.where(logits == m1, eidx, e), axis=1, keepdims=True)
        masked = jnp.where(eidx == i1, -jnp.inf, logits)
        m2 = jnp.max(masked, axis=1, keepdims=True)
        i2 = jnp.min(jnp.where(masked == m2, eidx, e), axis=1, keepdims=True)
        z = jnp.exp(m2 - m1)
        w0_ref[...] = 1.0 / (1.0 + z)
        w1_ref[...] = z / (1.0 + z)

        a1 = (eidx == i1).astype(jnp.bfloat16)
        a2 = (eidx == i2).astype(jnp.bfloat16)
        a = a1 + a2

        ri = lax.broadcasted_iota(jnp.int32, (SB, SB), 0)
        ci = lax.broadcasted_iota(jnp.int32, (SB, SB), 1)
        tri = (ci <= ri).astype(jnp.bfloat16)
        carry = jnp.zeros((1, e), jnp.float32)
        chunks = []
        for s in range(n // SB):
            cs = jnp.dot(tri, a[s * SB:(s + 1) * SB],
                         preferred_element_type=jnp.float32) + carry
            chunks.append(cs)
            carry = cs[SB - 1:SB, :]
        cincl = jnp.concatenate(chunks, axis=0)
        counts = carry
        cexcl = cincl - a.astype(jnp.float32)

        padded = jnp.floor((counts + (BM - 1)) * (1.0 / BM)) * BM
        triu = (lax.broadcasted_iota(jnp.int32, (e, e), 0)
                <= lax.broadcasted_iota(jnp.int32, (e, e), 1)).astype(jnp.float32)
        pend = jnp.dot(padded, triu, preferred_element_type=jnp.float32)
        starts = pend - padded

        a1f = a1.astype(jnp.float32)
        a2f = a2.astype(jnp.float32)
        rank0 = jnp.sum(a1f * cexcl, axis=1, keepdims=True)
        rank1 = jnp.sum(a2f * cexcl, axis=1, keepdims=True)
        rs0 = jnp.sum(a1f * starts, axis=1, keepdims=True)
        rs1 = jnp.sum(a2f * starts, axis=1, keepdims=True)
        row0_ref[...] = (rank0 + rs0).astype(jnp.int32)
        row1_ref[...] = (rank1 + rs1).astype(jnp.int32)

        bs = (lax.broadcasted_iota(jnp.int32, (nb, e), 0) * BM).astype(jnp.float32)
        pendb = jnp.broadcast_to(pend, (nb, e))
        be = jnp.minimum(jnp.sum((pendb <= bs).astype(jnp.int32),
                                 axis=1, keepdims=True), e - 1)
        oh_be = (lax.broadcasted_iota(jnp.int32, (nb, e), 1) == be
                 ).astype(jnp.float32)
        vend = jnp.sum(oh_be * (starts + counts), axis=1, keepdims=True)
        bs1 = bs[:, :1]
        nv = jnp.clip(vend - bs1, 0.0, float(BM))
        be_ref[...] = be
        nv_ref[...] = nv.astype(jnp.int32)

    return body


def _ffn_body(be_ref, nv_ref, xg_ref, w1_ref, b1_ref, w2_ref, b2_ref, y_ref):
    i = pl.program_id(0)

    @pl.when(nv_ref[i] > 0)
    def _compute():
        xb = xg_ref[...].astype(jnp.bfloat16)
        h = jnp.dot(xb, w1_ref[0], preferred_element_type=jnp.float32)
        h = h + b1_ref[0]
        h = 0.5 * h * (1.0 + lax.erf(h * _INV_SQRT2))
        acc = jnp.dot(h.astype(jnp.bfloat16), w2_ref[0],
                      preferred_element_type=jnp.float32)
        y_ref[...] = acc + b2_ref[0]


def _make_dispatch(n, d, r):
    tokw = n // NW
    mesh = plsc.VectorSubcoreMesh(core_axis_name="c", subcore_axis_name="s")

    @functools.partial(
        pl.kernel, mesh=mesh,
        out_type=jax.ShapeDtypeStruct((r, d), jnp.float32),
        scratch_types=[
            pltpu.VMEM((CH, d), jnp.float32),
            pltpu.VMEM((CH,), jnp.int32),
            pltpu.VMEM((CH,), jnp.int32),
            pltpu.SemaphoreType.DMA,
        ],
    )
    def dispatch(x_hbm, r0_hbm, r1_hbm, xg_hbm, xbuf, i0, i1, sem):
        w = lax.axis_index("s") * NC + lax.axis_index("c")
        base = w * tokw
        for c in range(tokw // CH):
            off = base + c * CH
            pltpu.sync_copy(x_hbm.at[pl.ds(off, CH)], xbuf)
            pltpu.sync_copy(r0_hbm.at[pl.ds(off, CH)], i0)
            pltpu.sync_copy(r1_hbm.at[pl.ds(off, CH)], i1)
            pltpu.async_copy(xbuf, xg_hbm.at[i0], sem).wait()
            pltpu.async_copy(xbuf, xg_hbm.at[i1], sem).wait()

    return dispatch


def _make_combine(n, d):
    tokw = n // NW
    nsl = d // 16
    mesh = plsc.VectorSubcoreMesh(core_axis_name="c", subcore_axis_name="s")

    @functools.partial(
        pl.kernel, mesh=mesh,
        out_type=jax.ShapeDtypeStruct((n, d), jnp.float32),
        scratch_types=[
            pltpu.VMEM((CH2, d), jnp.float32),
            pltpu.VMEM((CH2, d), jnp.float32),
            pltpu.VMEM((CH2,), jnp.int32),
            pltpu.VMEM((CH2,), jnp.int32),
            pltpu.VMEM((CH2, 16), jnp.float32),
            pltpu.VMEM((CH2, 16), jnp.float32),
            pltpu.SemaphoreType.DMA,
            pltpu.SemaphoreType.DMA,
        ],
    )
    def combine(y_hbm, r0_hbm, r1_hbm, w0_hbm, w1_hbm, out_hbm,
                yb0, yb1, i0, i1, wv0, wv1, s0, s1):
        w = lax.axis_index("s") * NC + lax.axis_index("c")
        base = w * tokw
        for c in range(tokw // CH2):
            off = base + c * CH2
            pltpu.sync_copy(r0_hbm.at[pl.ds(off, CH2)], i0)
            pltpu.sync_copy(r1_hbm.at[pl.ds(off, CH2)], i1)
            pltpu.sync_copy(w0_hbm.at[pl.ds(off, CH2)], wv0)
            pltpu.sync_copy(w1_hbm.at[pl.ds(off, CH2)], wv1)
            cp0 = pltpu.async_copy(y_hbm.at[i0], yb0, s0)
            cp1 = pltpu.async_copy(y_hbm.at[i1], yb1, s1)
            cp0.wait()
            cp1.wait()
            for t in range(CH2):
                wa = wv0[t, pl.ds(0, 16)]
                wb = wv1[t, pl.ds(0, 16)]

                def add_body(k, _):
                    cc = k * 16
                    yb0[t, pl.ds(cc, 16)] = (yb0[t, pl.ds(cc, 16)] * wa
                                             + yb1[t, pl.ds(cc, 16)] * wb)
                    return 0

                lax.fori_loop(0, nsl, add_body, 0, unroll=8)
            pltpu.sync_copy(yb0, out_hbm.at[pl.ds(off, CH2)])

    return combine


def kernel(hidden_states, gate_w, W1, b1, W2, b2):
    B, S, D = hidden_states.shape
    E = gate_w.shape[1]
    FF = W1.shape[2]
    N = B * S
    NB = (N * K) // BM + E
    R = NB * BM

    x2d = hidden_states.reshape(N, D)

    row0, row1, w0, w1, be, nv = pl.pallas_call(
        _make_gate_body(N, E, NB),
        out_shape=[jax.ShapeDtypeStruct((N, 1), jnp.int32),
                   jax.ShapeDtypeStruct((N, 1), jnp.int32),
                   jax.ShapeDtypeStruct((N, 1), jnp.float32),
                   jax.ShapeDtypeStruct((N, 1), jnp.float32),
                   jax.ShapeDtypeStruct((NB, 1), jnp.int32),
                   jax.ShapeDtypeStruct((NB, 1), jnp.int32)],
    )(x2d, gate_w)
    row0 = row0.reshape(N)
    row1 = row1.reshape(N)
    w0 = w0.reshape(N)
    w1 = w1.reshape(N)
    be = be.reshape(NB)
    nv = nv.reshape(NB)

    xg = _make_dispatch(N, D, R)(x2d, row0, row1)

    w1b = W1.astype(jnp.bfloat16)
    w2b = W2.astype(jnp.bfloat16)
    grid_spec = pltpu.PrefetchScalarGridSpec(
        num_scalar_prefetch=2,
        grid=(NB,),
        in_specs=[
            pl.BlockSpec((BM, D), lambda i, be, nv: (i, 0)),
            pl.BlockSpec((1, D, FF), lambda i, be, nv: (be[i], 0, 0)),
            pl.BlockSpec((1, 1, FF), lambda i, be, nv: (be[i], 0, 0)),
            pl.BlockSpec((1, FF, D), lambda i, be, nv: (be[i], 0, 0)),
            pl.BlockSpec((1, 1, D), lambda i, be, nv: (be[i], 0, 0)),
        ],
        out_specs=pl.BlockSpec((BM, D), lambda i, be, nv: (i, 0)),
    )
    y = pl.pallas_call(
        _ffn_body,
        grid_spec=grid_spec,
        out_shape=jax.ShapeDtypeStruct((R, D), jnp.float32),
    )(be, nv, xg, w1b, b1.reshape(E, 1, FF), w2b, b2.reshape(E, 1, D))

    w0b = jnp.broadcast_to(w0.reshape(N, 1), (N, 16))
    w1b_g = jnp.broadcast_to(w1.reshape(N, 1), (N, 16))
    out = _make_combine(N, D)(y, row0, row1, w0b, w1b_g)
    return out.reshape(B, S, D)


# speedup vs baseline: 1.1343x; 1.1343x over previous
import functools

import jax
import jax.numpy as jnp
from jax import lax
from jax.experimental import pallas as pl
from jax.experimental.pallas import tpu as pltpu
from jax.experimental.pallas import tpu_sc as plsc

K = 2
BM = 256
SB = 256
NC = 2
NS = 16
NW = NC * NS
CH = 32
CH2 = 16

_INV_SQRT2 = 0.7071067811865476


def _make_gate_body(n, e, nb):
    def body(x_ref, gw_ref, row0_ref, row1_ref, w0_ref, w1_ref,
             be_ref, nv_ref):
        x = x_ref[...]
        logits = jnp.dot(x, gw_ref[...], preferred_element_type=jnp.float32)
        eidx = lax.broadcasted_iota(jnp.int32, (n, e), 1)
        m1 = jnp.max(logits, axis=1, keepdims=True)
        i1 = jnp.min(jnp.where(logits == m1, eidx, e), axis=1, keepdims=True)
        masked = jnp.where(eidx == i1, -jnp.inf, logits)
        m2 = jnp.max(masked, axis=1, keepdims=True)
        i2 = jnp.min(jnp.where(masked == m2, eidx, e), axis=1, keepdims=True)
        z = jnp.exp(m2 - m1)
        w0_ref[...] = 1.0 / (1.0 + z)
        w1_ref[...] = z / (1.0 + z)

        a1 = (eidx == i1).astype(jnp.bfloat16)
        a2 = (eidx == i2).astype(jnp.bfloat16)
        a = a1 + a2

        ri = lax.broadcasted_iota(jnp.int32, (SB, SB), 0)
        ci = lax.broadcasted_iota(jnp.int32, (SB, SB), 1)
        tri = (ci <= ri).astype(jnp.bfloat16)
        carry = jnp.zeros((1, e), jnp.float32)
        chunks = []
        for s in range(n // SB):
            cs = jnp.dot(tri, a[s * SB:(s + 1) * SB],
                         preferred_element_type=jnp.float32) + carry
            chunks.append(cs)
            carry = cs[SB - 1:SB, :]
        cincl = jnp.concatenate(chunks, axis=0)
        counts = carry
        cexcl = cincl - a.astype(jnp.float32)

        padded = jnp.floor((counts + (BM - 1)) * (1.0 / BM)) * BM
        triu = (lax.broadcasted_iota(jnp.int32, (e, e), 0)
                <= lax.broadcasted_iota(jnp.int32, (e, e), 1)).astype(jnp.float32)
        pend = jnp.dot(padded, triu, preferred_element_type=jnp.float32)
        starts = pend - padded

        a1f = a1.astype(jnp.float32)
        a2f = a2.astype(jnp.float32)
        rank0 = jnp.sum(a1f * cexcl, axis=1, keepdims=True)
        rank1 = jnp.sum(a2f * cexcl, axis=1, keepdims=True)
        rs0 = jnp.sum(a1f * starts, axis=1, keepdims=True)
        rs1 = jnp.sum(a2f * starts, axis=1, keepdims=True)
        row0_ref[...] = (rank0 + rs0).astype(jnp.int32)
        row1_ref[...] = (rank1 + rs1).astype(jnp.int32)

        bs = (lax.broadcasted_iota(jnp.int32, (nb, e), 0) * BM).astype(jnp.float32)
        pendb = jnp.broadcast_to(pend, (nb, e))
        be = jnp.minimum(jnp.sum((pendb <= bs).astype(jnp.int32),
                                 axis=1, keepdims=True), e - 1)
        oh_be = (lax.broadcasted_iota(jnp.int32, (nb, e), 1) == be
                 ).astype(jnp.float32)
        vend = jnp.sum(oh_be * (starts + counts), axis=1, keepdims=True)
        bs1 = bs[:, :1]
        nv = jnp.clip(vend - bs1, 0.0, float(BM))
        be_ref[...] = be
        nv_ref[...] = nv.astype(jnp.int32)

    return body


def _ffn1_body(be_ref, nv_ref, xg_ref, w1_ref, b1_ref, h_ref, w1b_ref):
    i = pl.program_id(0)
    e = be_ref[i]
    pe = be_ref[jnp.maximum(i - 1, 0)]

    @pl.when((i == 0) | (e != pe))
    def _cast():
        w1b_ref[...] = w1_ref[0].astype(jnp.bfloat16)

    @pl.when(nv_ref[i] > 0)
    def _compute():
        xb = xg_ref[...].astype(jnp.bfloat16)
        h = jnp.dot(xb, w1b_ref[...], preferred_element_type=jnp.float32)
        h = h + b1_ref[0]
        h = 0.5 * h * (1.0 + lax.erf(h * _INV_SQRT2))
        h_ref[...] = h.astype(jnp.bfloat16)


def _ffn2_body(be_ref, nv_ref, h_ref, w2_ref, b2_ref, y_ref, w2b_ref):
    i = pl.program_id(0)
    e = be_ref[i]
    pe = be_ref[jnp.maximum(i - 1, 0)]

    @pl.when((i == 0) | (e != pe))
    def _cast():
        w2b_ref[...] = w2_ref[0].astype(jnp.bfloat16)

    @pl.when(nv_ref[i] > 0)
    def _compute():
        acc = jnp.dot(h_ref[...], w2b_ref[...],
                      preferred_element_type=jnp.float32)
        y_ref[...] = acc + b2_ref[0]


def _make_dispatch(n, d, r):
    tokw = n // NW
    mesh = plsc.VectorSubcoreMesh(core_axis_name="c", subcore_axis_name="s")

    @functools.partial(
        pl.kernel, mesh=mesh,
        out_type=jax.ShapeDtypeStruct((r, d), jnp.float32),
        scratch_types=[
            pltpu.VMEM((CH, d), jnp.float32),
            pltpu.VMEM((CH,), jnp.int32),
            pltpu.VMEM((CH,), jnp.int32),
            pltpu.SemaphoreType.DMA,
        ],
    )
    def dispatch(x_hbm, r0_hbm, r1_hbm, xg_hbm, xbuf, i0, i1, sem):
        w = lax.axis_index("s") * NC + lax.axis_index("c")
        base = w * tokw
        for c in range(tokw // CH):
            off = base + c * CH
            pltpu.sync_copy(x_hbm.at[pl.ds(off, CH)], xbuf)
            pltpu.sync_copy(r0_hbm.at[pl.ds(off, CH)], i0)
            pltpu.sync_copy(r1_hbm.at[pl.ds(off, CH)], i1)
            pltpu.async_copy(xbuf, xg_hbm.at[i0], sem).wait()
            pltpu.async_copy(xbuf, xg_hbm.at[i1], sem).wait()

    return dispatch


def _make_combine(n, d):
    tokw = n // NW
    nsl = d // 16
    mesh = plsc.VectorSubcoreMesh(core_axis_name="c", subcore_axis_name="s")

    @functools.partial(
        pl.kernel, mesh=mesh,
        out_type=jax.ShapeDtypeStruct((n, d), jnp.float32),
        scratch_types=[
            pltpu.VMEM((CH2, d), jnp.float32),
            pltpu.VMEM((CH2, d), jnp.float32),
            pltpu.VMEM((CH2,), jnp.int32),
            pltpu.VMEM((CH2,), jnp.int32),
            pltpu.VMEM((CH2, 16), jnp.float32),
            pltpu.VMEM((CH2, 16), jnp.float32),
            pltpu.SemaphoreType.DMA,
            pltpu.SemaphoreType.DMA,
        ],
    )
    def combine(y_hbm, r0_hbm, r1_hbm, w0_hbm, w1_hbm, out_hbm,
                yb0, yb1, i0, i1, wv0, wv1, s0, s1):
        w = lax.axis_index("s") * NC + lax.axis_index("c")
        base = w * tokw
        for c in range(tokw // CH2):
            off = base + c * CH2
            pltpu.sync_copy(r0_hbm.at[pl.ds(off, CH2)], i0)
            pltpu.sync_copy(r1_hbm.at[pl.ds(off, CH2)], i1)
            pltpu.sync_copy(w0_hbm.at[pl.ds(off, CH2)], wv0)
            pltpu.sync_copy(w1_hbm.at[pl.ds(off, CH2)], wv1)
            cp0 = pltpu.async_copy(y_hbm.at[i0], yb0, s0)
            cp1 = pltpu.async_copy(y_hbm.at[i1], yb1, s1)
            cp0.wait()
            cp1.wait()
            for t in range(CH2):
                wa = wv0[t, pl.ds(0, 16)]
                wb = wv1[t, pl.ds(0, 16)]

                def add_body(k, _):
                    cc = k * 16
                    yb0[t, pl.ds(cc, 16)] = (yb0[t, pl.ds(cc, 16)] * wa
                                             + yb1[t, pl.ds(cc, 16)] * wb)
                    return 0

                lax.fori_loop(0, nsl, add_body, 0, unroll=8)
            pltpu.sync_copy(yb0, out_hbm.at[pl.ds(off, CH2)])

    return combine


def kernel(hidden_states, gate_w, W1, b1, W2, b2):
    B, S, D = hidden_states.shape
    E = gate_w.shape[1]
    FF = W1.shape[2]
    N = B * S
    NB = (N * K) // BM + E
    R = NB * BM

    x2d = hidden_states.reshape(N, D)

    row0, row1, w0, w1, be, nv = pl.pallas_call(
        _make_gate_body(N, E, NB),
        out_shape=[jax.ShapeDtypeStruct((N, 1), jnp.int32),
                   jax.ShapeDtypeStruct((N, 1), jnp.int32),
                   jax.ShapeDtypeStruct((N, 1), jnp.float32),
                   jax.ShapeDtypeStruct((N, 1), jnp.float32),
                   jax.ShapeDtypeStruct((NB, 1), jnp.int32),
                   jax.ShapeDtypeStruct((NB, 1), jnp.int32)],
    )(x2d, gate_w)
    row0 = row0.reshape(N)
    row1 = row1.reshape(N)
    w0 = w0.reshape(N)
    w1 = w1.reshape(N)
    be = be.reshape(NB)
    nv = nv.reshape(NB)

    xg = _make_dispatch(N, D, R)(x2d, row0, row1)

    grid_spec1 = pltpu.PrefetchScalarGridSpec(
        num_scalar_prefetch=2,
        grid=(NB,),
        in_specs=[
            pl.BlockSpec((BM, D), lambda i, be, nv: (i, 0)),
            pl.BlockSpec((1, D, FF), lambda i, be, nv: (be[i], 0, 0)),
            pl.BlockSpec((1, 1, FF), lambda i, be, nv: (be[i], 0, 0)),
        ],
        out_specs=pl.BlockSpec((BM, FF), lambda i, be, nv: (i, 0)),
        scratch_shapes=[pltpu.VMEM((D, FF), jnp.bfloat16)],
    )
    h = pl.pallas_call(
        _ffn1_body,
        grid_spec=grid_spec1,
        out_shape=jax.ShapeDtypeStruct((R, FF), jnp.bfloat16),
    )(be, nv, xg, W1, b1.reshape(E, 1, FF))

    grid_spec2 = pltpu.PrefetchScalarGridSpec(
        num_scalar_prefetch=2,
        grid=(NB,),
        in_specs=[
            pl.BlockSpec((BM, FF), lambda i, be, nv: (i, 0)),
            pl.BlockSpec((1, FF, D), lambda i, be, nv: (be[i], 0, 0)),
            pl.BlockSpec((1, 1, D), lambda i, be, nv: (be[i], 0, 0)),
        ],
        out_specs=pl.BlockSpec((BM, D), lambda i, be, nv: (i, 0)),
        scratch_shapes=[pltpu.VMEM((FF, D), jnp.bfloat16)],
    )
    y = pl.pallas_call(
        _ffn2_body,
        grid_spec=grid_spec2,
        out_shape=jax.ShapeDtypeStruct((R, D), jnp.float32),
    )(be, nv, h, W2, b2.reshape(E, 1, D))

    w0b = jnp.broadcast_to(w0.reshape(N, 1), (N, 16))
    w1b_g = jnp.broadcast_to(w1.reshape(N, 1), (N, 16))
    out = _make_combine(N, D)(y, row0, row1, w0b, w1b_g)
    return out.reshape(B, S, D)


# speedup vs baseline: 1.2076x; 1.0645x over previous
import functools

import jax
import jax.numpy as jnp
from jax import lax
from jax.experimental import pallas as pl
from jax.experimental.pallas import tpu as pltpu
from jax.experimental.pallas import tpu_sc as plsc

K = 2
BM = 256
SB = 256
NC = 2
NS = 16
NW = NC * NS
CH = 32
CH2 = 16

_INV_SQRT2 = 0.7071067811865476


def _make_gate_body(n, e, nb):
    def body(x_ref, gw_ref, row0_ref, row1_ref, w0_ref, w1_ref,
             be_ref, nv_ref):
        x = x_ref[...]
        logits = jnp.dot(x, gw_ref[...], preferred_element_type=jnp.float32)
        eidx = lax.broadcasted_iota(jnp.int32, (n, e), 1)
        m1 = jnp.max(logits, axis=1, keepdims=True)
        i1 = jnp.min(jnp.where(logits == m1, eidx, e), axis=1, keepdims=True)
        masked = jnp.where(eidx == i1, -jnp.inf, logits)
        m2 = jnp.max(masked, axis=1, keepdims=True)
        i2 = jnp.min(jnp.where(masked == m2, eidx, e), axis=1, keepdims=True)
        z = jnp.exp(m2 - m1)
        w0_ref[...] = 1.0 / (1.0 + z)
        w1_ref[...] = z / (1.0 + z)

        a1 = (eidx == i1).astype(jnp.bfloat16)
        a2 = (eidx == i2).astype(jnp.bfloat16)
        a = a1 + a2

        ri = lax.broadcasted_iota(jnp.int32, (SB, SB), 0)
        ci = lax.broadcasted_iota(jnp.int32, (SB, SB), 1)
        tri = (ci <= ri).astype(jnp.bfloat16)
        carry = jnp.zeros((1, e), jnp.float32)
        chunks = []
        for s in range(n // SB):
            cs = jnp.dot(tri, a[s * SB:(s + 1) * SB],
                         preferred_element_type=jnp.float32) + carry
            chunks.append(cs)
            carry = cs[SB - 1:SB, :]
        cincl = jnp.concatenate(chunks, axis=0)
        counts = carry
        cexcl = cincl - a.astype(jnp.float32)

        padded = jnp.floor((counts + (BM - 1)) * (1.0 / BM)) * BM
        triu = (lax.broadcasted_iota(jnp.int32, (e, e), 0)
                <= lax.broadcasted_iota(jnp.int32, (e, e), 1)).astype(jnp.float32)
        pend = jnp.dot(padded, triu, preferred_element_type=jnp.float32)
        starts = pend - padded

        a1f = a1.astype(jnp.float32)
        a2f = a2.astype(jnp.float32)
        rank0 = jnp.sum(a1f * cexcl, axis=1, keepdims=True)
        rank1 = jnp.sum(a2f * cexcl, axis=1, keepdims=True)
        rs0 = jnp.sum(a1f * starts, axis=1, keepdims=True)
        rs1 = jnp.sum(a2f * starts, axis=1, keepdims=True)
        row0_ref[...] = (rank0 + rs0).astype(jnp.int32)
        row1_ref[...] = (rank1 + rs1).astype(jnp.int32)

        bs = (lax.broadcasted_iota(jnp.int32, (nb, e), 0) * BM).astype(jnp.float32)
        pendb = jnp.broadcast_to(pend, (nb, e))
        be = jnp.minimum(jnp.sum((pendb <= bs).astype(jnp.int32),
                                 axis=1, keepdims=True), e - 1)
        oh_be = (lax.broadcasted_iota(jnp.int32, (nb, e), 1) == be
                 ).astype(jnp.float32)
        vend = jnp.sum(oh_be * (starts + counts), axis=1, keepdims=True)
        bs1 = bs[:, :1]
        nv = jnp.clip(vend - bs1, 0.0, float(BM))
        be_ref[...] = be
        nv_ref[...] = nv.astype(jnp.int32)

    return body


def _ffn1_body(be_ref, nv_ref, xg_ref, w1_ref, b1_ref, h_ref, w1b_ref):
    i = pl.program_id(0)
    e = be_ref[i]
    pe = be_ref[jnp.maximum(i - 1, 0)]

    @pl.when((i == 0) | (e != pe))
    def _cast():
        w1b_ref[...] = w1_ref[0].astype(jnp.bfloat16)

    @pl.when(nv_ref[i] > 0)
    def _compute():
        xb = xg_ref[...].astype(jnp.bfloat16)
        h = jnp.dot(xb, w1b_ref[...], preferred_element_type=jnp.float32)
        h = h + b1_ref[0]
        h = 0.5 * h * (1.0 + lax.erf(h * _INV_SQRT2))
        h_ref[...] = h.astype(jnp.bfloat16)


def _ffn2_body(be_ref, nv_ref, h_ref, w2_ref, b2_ref, wg_ref, y_ref, w2b_ref):
    i = pl.program_id(0)
    e = be_ref[i]
    pe = be_ref[jnp.maximum(i - 1, 0)]

    @pl.when((i == 0) | (e != pe))
    def _cast():
        w2b_ref[...] = w2_ref[0].astype(jnp.bfloat16)

    @pl.when(nv_ref[i] > 0)
    def _compute():
        acc = jnp.dot(h_ref[...], w2b_ref[...],
                      preferred_element_type=jnp.float32)
        y_ref[...] = (acc + b2_ref[0]) * wg_ref[:, :1]


def _make_dispatch(n, d, r):
    tokw = n // NW
    mesh = plsc.VectorSubcoreMesh(core_axis_name="c", subcore_axis_name="s")

    @functools.partial(
        pl.kernel, mesh=mesh,
        out_type=[jax.ShapeDtypeStruct((r, d), jnp.float32),
                  jax.ShapeDtypeStruct((r, 128), jnp.float32)],
        scratch_types=[
            pltpu.VMEM((CH, d), jnp.float32),
            pltpu.VMEM((CH, 128), jnp.float32),
            pltpu.VMEM((CH, 128), jnp.float32),
            pltpu.VMEM((CH,), jnp.int32),
            pltpu.VMEM((CH,), jnp.int32),
            pltpu.SemaphoreType.DMA,
        ],
    )
    def dispatch(x_hbm, w0_hbm, w1_hbm, r0_hbm, r1_hbm, xg_hbm, wg_hbm,
                 xbuf, wbuf0, wbuf1, i0, i1, sem):
        w = lax.axis_index("s") * NC + lax.axis_index("c")
        base = w * tokw
        for c in range(tokw // CH):
            off = base + c * CH
            pltpu.sync_copy(x_hbm.at[pl.ds(off, CH)], xbuf)
            pltpu.sync_copy(w0_hbm.at[pl.ds(off, CH)], wbuf0)
            pltpu.sync_copy(w1_hbm.at[pl.ds(off, CH)], wbuf1)
            pltpu.sync_copy(r0_hbm.at[pl.ds(off, CH)], i0)
            pltpu.sync_copy(r1_hbm.at[pl.ds(off, CH)], i1)
            cps = [pltpu.async_copy(xbuf, xg_hbm.at[i0], sem),
                   pltpu.async_copy(xbuf, xg_hbm.at[i1], sem),
                   pltpu.async_copy(wbuf0, wg_hbm.at[i0], sem),
                   pltpu.async_copy(wbuf1, wg_hbm.at[i1], sem)]
            for cp in cps:
                cp.wait()

    return dispatch


def _make_combine(n, d):
    tokw = n // NW
    nsl = d // 16
    mesh = plsc.VectorSubcoreMesh(core_axis_name="c", subcore_axis_name="s")

    @functools.partial(
        pl.kernel, mesh=mesh,
        out_type=jax.ShapeDtypeStruct((n, d), jnp.float32),
        scratch_types=[
            pltpu.VMEM((CH2, d), jnp.float32),
            pltpu.VMEM((CH2, d), jnp.float32),
            pltpu.VMEM((CH2,), jnp.int32),
            pltpu.VMEM((CH2,), jnp.int32),
            pltpu.SemaphoreType.DMA,
            pltpu.SemaphoreType.DMA,
        ],
    )
    def combine(y_hbm, r0_hbm, r1_hbm, out_hbm, yb0, yb1, i0, i1, s0, s1):
        w = lax.axis_index("s") * NC + lax.axis_index("c")
        base = w * tokw
        for c in range(tokw // CH2):
            off = base + c * CH2
            pltpu.sync_copy(r0_hbm.at[pl.ds(off, CH2)], i0)
            pltpu.sync_copy(r1_hbm.at[pl.ds(off, CH2)], i1)
            cp0 = pltpu.async_copy(y_hbm.at[i0], yb0, s0)
            cp1 = pltpu.async_copy(y_hbm.at[i1], yb1, s1)
            cp0.wait()
            cp1.wait()
            for t in range(CH2):
                @plsc.parallel_loop(0, nsl, 1, unroll=8)
                def _add(k):
                    cc = k * 16
                    yb0[t, pl.ds(cc, 16)] = (yb0[t, pl.ds(cc, 16)]
                                             + yb1[t, pl.ds(cc, 16)])
            pltpu.sync_copy(yb0, out_hbm.at[pl.ds(off, CH2)])

    return combine


def kernel(hidden_states, gate_w, W1, b1, W2, b2):
    B, S, D = hidden_states.shape
    E = gate_w.shape[1]
    FF = W1.shape[2]
    N = B * S
    NB = (N * K) // BM + E
    R = NB * BM

    x2d = hidden_states.reshape(N, D)

    row0, row1, w0, w1, be, nv = pl.pallas_call(
        _make_gate_body(N, E, NB),
        out_shape=[jax.ShapeDtypeStruct((N, 1), jnp.int32),
                   jax.ShapeDtypeStruct((N, 1), jnp.int32),
                   jax.ShapeDtypeStruct((N, 1), jnp.float32),
                   jax.ShapeDtypeStruct((N, 1), jnp.float32),
                   jax.ShapeDtypeStruct((NB, 1), jnp.int32),
                   jax.ShapeDtypeStruct((NB, 1), jnp.int32)],
    )(x2d, gate_w)
    row0 = row0.reshape(N)
    row1 = row1.reshape(N)
    w0 = w0.reshape(N)
    w1 = w1.reshape(N)
    be = be.reshape(NB)
    nv = nv.reshape(NB)

    w0b = jnp.broadcast_to(w0.reshape(N, 1), (N, 128))
    w1b = jnp.broadcast_to(w1.reshape(N, 1), (N, 128))
    xg, wg = _make_dispatch(N, D, R)(x2d, w0b, w1b, row0, row1)

    grid_spec1 = pltpu.PrefetchScalarGridSpec(
        num_scalar_prefetch=2,
        grid=(NB,),
        in_specs=[
            pl.BlockSpec((BM, D), lambda i, be, nv: (i, 0)),
            pl.BlockSpec((1, D, FF), lambda i, be, nv: (be[i], 0, 0)),
            pl.BlockSpec((1, 1, FF), lambda i, be, nv: (be[i], 0, 0)),
        ],
        out_specs=pl.BlockSpec((BM, FF), lambda i, be, nv: (i, 0)),
        scratch_shapes=[pltpu.VMEM((D, FF), jnp.bfloat16)],
    )
    h = pl.pallas_call(
        _ffn1_body,
        grid_spec=grid_spec1,
        out_shape=jax.ShapeDtypeStruct((R, FF), jnp.bfloat16),
    )(be, nv, xg, W1, b1.reshape(E, 1, FF))

    grid_spec2 = pltpu.PrefetchScalarGridSpec(
        num_scalar_prefetch=2,
        grid=(NB,),
        in_specs=[
            pl.BlockSpec((BM, FF), lambda i, be, nv: (i, 0)),
            pl.BlockSpec((1, FF, D), lambda i, be, nv: (be[i], 0, 0)),
            pl.BlockSpec((1, 1, D), lambda i, be, nv: (be[i], 0, 0)),
            pl.BlockSpec((BM, 128), lambda i, be, nv: (i, 0)),
        ],
        out_specs=pl.BlockSpec((BM, D), lambda i, be, nv: (i, 0)),
        scratch_shapes=[pltpu.VMEM((FF, D), jnp.bfloat16)],
    )
    y = pl.pallas_call(
        _ffn2_body,
        grid_spec=grid_spec2,
        out_shape=jax.ShapeDtypeStruct((R, D), jnp.float32),
    )(be, nv, h, W2, b2.reshape(E, 1, D), wg)

    out = _make_combine(N, D)(y, row0, row1)
    return out.reshape(B, S, D)
